# Initial kernel scaffold; baseline (speedup 1.0000x reference)
import jax
import jax.numpy as jnp
from jax.experimental import pallas as pl


def kernel(x, shared_fc, shared_proj, routed_fc, routed_proj, centroids, routing_biases):
    raise NotImplementedError("write your pallas kernel here")


# speedup vs baseline: 3.3966x; 3.3966x over previous
import functools

import jax
import jax.numpy as jnp
from jax import lax
from jax.experimental import pallas as pl

B, S, D = 1, 2048, 768
E, K, NS = 8, 2, 2
H = 4 * D
NE = E + NS
TB = 256
NSB = S // TB


def _gelu(h):
    return 0.5 * h * (1.0 + lax.erf(h * 0.7071067811865476))


def _router_body(x_ref, c_ref, b_ref, w_ref):
    x = x_ref[...]
    c = c_ref[...]
    raw = lax.dot_general(x, c, (((1,), (1,)), ((), ())),
                          preferred_element_type=jnp.float32,
                          precision=lax.Precision.DEFAULT)
    sbal = jax.nn.sigmoid(raw + b_ref[...])
    lane8 = lax.broadcasted_iota(jnp.int32, (S, E), 1)
    m1 = jnp.max(sbal, axis=1, keepdims=True)
    i0 = jnp.min(jnp.where(sbal == m1, lane8, E), axis=1, keepdims=True)
    neg = jnp.where(lane8 == i0, -1.0, sbal)
    m2 = jnp.max(neg, axis=1, keepdims=True)
    i1 = jnp.min(jnp.where(neg == m2, lane8, E), axis=1, keepdims=True)
    sg = jax.nn.sigmoid(raw)
    g0 = jnp.sum(jnp.where(lane8 == i0, sg, 0.0), axis=1, keepdims=True)
    g1 = jnp.sum(jnp.where(lane8 == i1, sg, 0.0), axis=1, keepdims=True)
    p0 = jax.nn.sigmoid(g0 - g1)
    p1 = 1.0 - p0
    lane16 = lax.broadcasted_iota(jnp.int32, (S, 16), 1)
    w = (p0 * (lane16 == i0) + p1 * (lane16 == i1)
         + (lane16 == E).astype(jnp.float32)
         + (lane16 == E + 1).astype(jnp.float32))
    w_ref[...] = w


def _ffn_body(x_ref, fc_ref, proj_ref, w_ref, o_ref):
    e = pl.program_id(0)
    sb = pl.program_id(1)
    xb = x_ref[...]
    h = lax.dot_general(xb.astype(jnp.bfloat16), fc_ref[0],
                        (((1,), (1,)), ((), ())),
                        preferred_element_type=jnp.float32)
    h = _gelu(h)
    y = lax.dot_general(h.astype(jnp.bfloat16), proj_ref[0],
                        (((1,), (1,)), ((), ())),
                        preferred_element_type=jnp.float32)
    lane16 = lax.broadcasted_iota(jnp.int32, (TB, 16), 1)
    wcol = jnp.sum(jnp.where(lane16 == e, w_ref[...], 0.0), axis=1,
                   keepdims=True)
    contrib = y * wcol
    rows = pl.ds(sb * TB, TB)

    @pl.when(e == 0)
    def _():
        o_ref[rows, :] = xb + contrib

    @pl.when(e != 0)
    def _():
        o_ref[rows, :] += contrib


def kernel(x, shared_fc, shared_proj, routed_fc, routed_proj, centroids,
           routing_biases):
    x2 = x.reshape(S, D)
    fc_all = jnp.concatenate([routed_fc, shared_fc], axis=0).astype(jnp.bfloat16)
    proj_all = jnp.concatenate([routed_proj, shared_proj], axis=0).astype(jnp.bfloat16)
    bias2d = routing_biases.reshape(1, E)

    w = pl.pallas_call(
        _router_body,
        out_shape=jax.ShapeDtypeStruct((S, 16), jnp.float32),
    )(x2, centroids, bias2d)

    out = pl.pallas_call(
        _ffn_body,
        grid=(NE, NSB),
        in_specs=[
            pl.BlockSpec((TB, D), lambda e, sb: (sb, 0)),
            pl.BlockSpec((1, H, D), lambda e, sb: (e, 0, 0)),
            pl.BlockSpec((1, D, H), lambda e, sb: (e, 0, 0)),
            pl.BlockSpec((TB, 16), lambda e, sb: (sb, 0)),
        ],
        out_specs=pl.BlockSpec((S, D), lambda e, sb: (0, 0)),
        out_shape=jax.ShapeDtypeStruct((S, D), jnp.float32),
    )(x2, fc_all, proj_all, w)

    return out.reshape(B, S, D)
